# baseline (speedup 1.0000x reference)
import functools

import jax
import jax.numpy as jnp
from jax import lax
from jax.experimental import pallas as pl
from jax.experimental.pallas import tpu as pltpu
from jax.experimental.pallas import tpu_sc as plsc

N = 10000
E = 320000
H = 128
MNEG = 1000.0

NC = 2
NS = 16
NW = NC * NS
L = 128
GPT = 80
EPAD = NW * GPT * L
NPAD = 10240
NSL = NPAD // NS
NBUF = 2
HGPT = GPT // 2
IGC = 40
PNBUF = 2


@functools.cache
def _sc_kernels():
    mesh = plsc.VectorSubcoreMesh(
        core_axis_name="c", subcore_axis_name="s", num_cores=NC, num_subcores=NS)
    deg = functools.partial(
        pl.kernel, mesh=mesh,
        out_type=jax.ShapeDtypeStruct((NC * NPAD,), jnp.float32),
        scratch_types=[
            pltpu.VMEM_SHARED((NPAD,), jnp.float32),
            pltpu.VMEM((GPT, L), jnp.int32),
            pltpu.VMEM((L,), jnp.float32),
            pltpu.VMEM((NSL,), jnp.float32),
        ],
    )(_sc_degree_body)
    seg = functools.partial(
        pl.kernel, mesh=mesh,
        out_type=jax.ShapeDtypeStruct((NC, NPAD, H), jnp.float32),
        scratch_types=[
            pltpu.VMEM_SHARED((NPAD, H), jnp.float32),
            pltpu.VMEM((IGC, L), jnp.int32),
            pltpu.VMEM((IGC, L), jnp.int32),
            [pltpu.VMEM((L, H), jnp.float32) for _ in range(NBUF)],
            [pltpu.SemaphoreType.DMA for _ in range(NBUF)],
        ],
    )(_sc_segsum_body)
    pair = lambda h: functools.partial(
        pl.kernel, mesh=mesh,
        out_type=jax.ShapeDtypeStruct((EPAD // 2, H), jnp.float32),
        scratch_types=[
            pltpu.VMEM((HGPT, L), jnp.int32),
            pltpu.VMEM((HGPT, L), jnp.int32),
            pltpu.VMEM((HGPT, L), jnp.int32),
            pltpu.VMEM((HGPT, L), jnp.int32),
            pltpu.VMEM((HGPT, L), jnp.int32),
            pltpu.VMEM((HGPT, L), jnp.int32),
            [pltpu.VMEM((L, H), jnp.float32) for _ in range(PNBUF)],
            [pltpu.VMEM((L, H), jnp.float32) for _ in range(PNBUF)],
            [pltpu.SemaphoreType.DMA for _ in range(PNBUF)],
            [pltpu.SemaphoreType.DMA for _ in range(PNBUF)],
        ],
        compiler_params=pltpu.CompilerParams(needs_layout_passes=False),
    )(functools.partial(_sc_pair_body, h))
    return deg, seg, pair(0), pair(1)


def _sc_degree_body(dst2d, zeros_n, ones_l, degp, dacc, didx, ones_v, vbuf):
    c = lax.axis_index("c")
    s = lax.axis_index("s")
    wid = c * NS + s
    pltpu.sync_copy(zeros_n.at[pl.ds(s * NSL, NSL)], vbuf)
    pltpu.sync_copy(vbuf, dacc.at[pl.ds(s * NSL, NSL)])
    pltpu.sync_copy(ones_l, ones_v)
    pltpu.sync_copy(dst2d.at[pl.ds(wid * GPT, GPT)], didx)
    plsc.subcore_barrier()

    @pl.loop(0, GPT)
    def _(j):
        pltpu.sync_copy(ones_v, dacc.at[didx.at[j]], add=True)

    plsc.subcore_barrier()
    pltpu.sync_copy(dacc.at[pl.ds(s * NSL, NSL)], vbuf)
    pltpu.sync_copy(vbuf, degp.at[pl.ds(c * NPAD + s * NSL, NSL)])


def _sc_segsum_body(y, src2d, dst2d, zeros_nh, acc_out, acc, sidx, didx, rows, sems):
    c = lax.axis_index("c")
    s = lax.axis_index("s")
    wid = c * NS + s
    pltpu.sync_copy(zeros_nh.at[pl.ds(s * NSL, NSL)], acc.at[pl.ds(s * NSL, NSL)])
    plsc.subcore_barrier()

    @pl.loop(0, GPT // IGC)
    def _(ic):
        base = wid * GPT + ic * IGC
        pltpu.sync_copy(src2d.at[pl.ds(base, IGC)], sidx)
        pltpu.sync_copy(dst2d.at[pl.ds(base, IGC)], didx)
        for b in range(NBUF):
            pltpu.async_copy(y.at[sidx.at[b]], rows[b], sems[b])

        @pl.loop(0, IGC, step=NBUF)
        def _(j):
            for b in range(NBUF):
                jj = j + b
                pltpu.make_async_copy(y.at[sidx.at[jj]], rows[b], sems[b]).wait()
                pltpu.sync_copy(rows[b], acc.at[didx.at[jj]], add=True)

                @pl.when(jj < IGC - NBUF)
                def _():
                    pltpu.async_copy(y.at[sidx.at[jj + NBUF]], rows[b], sems[b])

    plsc.subcore_barrier()
    pltpu.sync_copy(acc.at[pl.ds(s * NSL, NSL)], acc_out.at[c, pl.ds(s * NSL, NSL)])


def _sc_pair_body(h, a_tab, b_tab, src2d, dst2d, sel2d, trash2d, g_out,
                  sidx, didx, selv2, csrc, cdst, cpos,
                  bufa, bufb, sema, semb):
    c = lax.axis_index("c")
    s = lax.axis_index("s")
    wid = c * NS + s
    hbase = h * (NW * HGPT) + wid * HGPT
    pltpu.sync_copy(src2d.at[pl.ds(hbase, HGPT)], sidx)
    pltpu.sync_copy(dst2d.at[pl.ds(hbase, HGPT)], didx)
    pltpu.sync_copy(sel2d.at[pl.ds(hbase, HGPT)], selv2)
    pltpu.sync_copy(src2d.at[pl.ds(hbase, HGPT)], csrc)
    pltpu.sync_copy(dst2d.at[pl.ds(hbase, HGPT)], cdst)
    pltpu.sync_copy(trash2d, cpos)

    lanes = lax.iota(jnp.int32, 16)

    @pl.loop(0, HGPT, init_carry=jnp.zeros((16,), jnp.int32))
    def _compact(j, cnt):
        for k in range(L // 16):
            selv = selv2[j, pl.ds(k * 16, 16)]
            srcv = sidx[j, pl.ds(k * 16, 16)]
            dstv = didx[j, pl.ds(k * 16, 16)]
            posv = wid * (HGPT * L) + j * L + k * 16 + lanes
            m = selv != 0
            pref = plsc.cumsum(selv)
            dest = cnt + pref - 1
            dr = lax.shift_right_logical(dest, 7)
            dl = lax.bitwise_and(dest, 127)
            plsc.store_scatter(csrc, [dr, dl], srcv, mask=m)
            plsc.store_scatter(cdst, [dr, dl], dstv, mask=m)
            plsc.store_scatter(cpos, [dr, dl], posv, mask=m)
            cnt = cnt + plsc.all_reduce_population_count(m)
        return cnt

    cnt = _compact[0]
    gc = lax.shift_right_logical(cnt + 127, 7)

    for b in range(PNBUF):
        @pl.when(b < gc)
        def _():
            pltpu.async_copy(a_tab.at[csrc.at[b]], bufa[b], sema[b])
            pltpu.async_copy(b_tab.at[cdst.at[b]], bufb[b], semb[b])

    gc_pad = lax.bitwise_and(gc + (PNBUF - 1), jnp.int32(-PNBUF))

    @pl.loop(0, gc_pad, step=PNBUF)
    def _(j):
        for b in range(PNBUF):
            jj = j + b

            @pl.when(jj < gc)
            def _():
                pltpu.make_async_copy(a_tab.at[csrc.at[jj]], bufa[b], sema[b]).wait()
                pltpu.make_async_copy(b_tab.at[cdst.at[jj]], bufb[b], semb[b]).wait()

                @pl.loop(0, L, unroll=4)
                def _(r):
                    for k in range(H // 16):
                        plsc.addupdate(bufa[b].at[r, pl.ds(k * 16, 16)],
                                       bufb[b][r, pl.ds(k * 16, 16)])

                pltpu.sync_copy(bufa[b], g_out.at[cpos.at[jj]])

                @pl.when(jj + PNBUF < gc)
                def _():
                    pltpu.async_copy(a_tab.at[csrc.at[jj + PNBUF]], bufa[b], sema[b])
                    pltpu.async_copy(b_tab.at[cdst.at[jj + PNBUF]], bufb[b], semb[b])


def _tc_prep_kernel(x_ref, wg_ref, wp1_ref, war_ref, we_ref, be_ref, bar_ref,
                    bp1_ref, we2_ref, c0_ref, c1_ref, acc):
    i = pl.program_id(0)

    @pl.when(i == 0)
    def _():
        acc[...] = jnp.zeros_like(acc)

    acc[...] += jnp.sum(x_ref[...], axis=0, keepdims=True)

    @pl.when(i == pl.num_programs(0) - 1)
    def _():
        wc = war_ref[2 * H:, :]
        we2_ref[...] = jnp.dot(we_ref[...], wc, preferred_element_type=jnp.float32)
        c0_ref[...] = jnp.dot(be_ref[...], wc, preferred_element_type=jnp.float32) + bar_ref[...]
        g = jnp.dot(acc[...] * (1.0 / N), wg_ref[...], preferred_element_type=jnp.float32)
        w1ab = wp1_ref[:H, :] + wp1_ref[H:2 * H, :]
        c1_ref[...] = jnp.dot(g, w1ab, preferred_element_type=jnp.float32) + bp1_ref[...]


def _tc_prep(x, wg, wp1, war, we, be, bar, bp1):
    nb = 10
    bn = N // nb
    return pl.pallas_call(
        _tc_prep_kernel,
        grid=(nb,),
        in_specs=[
            pl.BlockSpec((bn, H), lambda i: (i, 0)),
            pl.BlockSpec((H, H), lambda i: (0, 0)),
            pl.BlockSpec((3 * H, H), lambda i: (0, 0)),
            pl.BlockSpec((3 * H, H), lambda i: (0, 0)),
            pl.BlockSpec((16, H), lambda i: (0, 0)),
            pl.BlockSpec((1, H), lambda i: (0, 0)),
            pl.BlockSpec((1, H), lambda i: (0, 0)),
            pl.BlockSpec((1, H), lambda i: (0, 0)),
        ],
        out_specs=[
            pl.BlockSpec((16, H), lambda i: (0, 0)),
            pl.BlockSpec((1, H), lambda i: (0, 0)),
            pl.BlockSpec((1, H), lambda i: (0, 0)),
        ],
        out_shape=[
            jax.ShapeDtypeStruct((16, H), jnp.float32),
            jax.ShapeDtypeStruct((1, H), jnp.float32),
            jax.ShapeDtypeStruct((1, H), jnp.float32),
        ],
        scratch_shapes=[pltpu.VMEM((1, H), jnp.float32)],
    )(x, wg, wp1, war, we, be, bar, bp1)


def _tc_first_kernel(x_ref, w_ref, degt_ref, y_ref, dinv_ref):
    deg = 1.0 + degt_ref[:, 0:1] + degt_ref[:, 1:2]
    dinv = lax.rsqrt(deg)
    dinv_ref[...] = dinv
    y_ref[...] = jnp.dot(x_ref[...], w_ref[...], preferred_element_type=jnp.float32) * dinv


def _tc_first(x, w0, degt):
    nb = 10
    bn = N // nb
    return pl.pallas_call(
        _tc_first_kernel,
        grid=(nb,),
        in_specs=[
            pl.BlockSpec((bn, H), lambda i: (i, 0)),
            pl.BlockSpec((H, H), lambda i: (0, 0)),
            pl.BlockSpec((bn, 2), lambda i: (i, 0)),
        ],
        out_specs=[
            pl.BlockSpec((bn, H), lambda i: (i, 0)),
            pl.BlockSpec((bn, 1), lambda i: (i, 0)),
        ],
        out_shape=[
            jax.ShapeDtypeStruct((N, H), jnp.float32),
            jax.ShapeDtypeStruct((N, 1), jnp.float32),
        ],
    )(x, w0, degt)


def _tc_layer_kernel(sp_ref, y_ref, dinv_ref, b_ref, w_ref, ynext_ref):
    s = sp_ref[0] + sp_ref[1]
    dinv = dinv_ref[...]
    out = jax.nn.relu(dinv * (s + y_ref[...]) + b_ref[...])
    ynext_ref[...] = jnp.dot(out, w_ref[...], preferred_element_type=jnp.float32) * dinv


def _tc_layer(sp, y, dinv, b, wnext):
    nb = 10
    bn = N // nb
    return pl.pallas_call(
        _tc_layer_kernel,
        grid=(nb,),
        in_specs=[
            pl.BlockSpec((2, bn, H), lambda i: (0, i, 0)),
            pl.BlockSpec((bn, H), lambda i: (i, 0)),
            pl.BlockSpec((bn, 1), lambda i: (i, 0)),
            pl.BlockSpec((1, H), lambda i: (0, 0)),
            pl.BlockSpec((H, H), lambda i: (0, 0)),
        ],
        out_specs=pl.BlockSpec((bn, H), lambda i: (i, 0)),
        out_shape=jax.ShapeDtypeStruct((N, H), jnp.float32),
    )(sp, y, dinv, b, wnext)


def _tc_last_kernel(sp_ref, y_ref, dinv_ref, b_ref, wa_ref, wb_ref, c0_ref,
                    a_ref, bt_ref):
    s = sp_ref[0] + sp_ref[1]
    out = jax.nn.relu(dinv_ref[...] * (s + y_ref[...]) + b_ref[...])
    a_ref[...] = jnp.dot(out, wa_ref[...], preferred_element_type=jnp.float32) + c0_ref[...]
    bt_ref[...] = jnp.dot(out, wb_ref[...], preferred_element_type=jnp.float32)


def _tc_last(sp, y, dinv, b, wa, wb, c0):
    nb = 10
    bn = N // nb
    return pl.pallas_call(
        _tc_last_kernel,
        grid=(nb,),
        in_specs=[
            pl.BlockSpec((2, bn, H), lambda i: (0, i, 0)),
            pl.BlockSpec((bn, H), lambda i: (i, 0)),
            pl.BlockSpec((bn, 1), lambda i: (i, 0)),
            pl.BlockSpec((1, H), lambda i: (0, 0)),
            pl.BlockSpec((H, H), lambda i: (0, 0)),
            pl.BlockSpec((H, H), lambda i: (0, 0)),
            pl.BlockSpec((1, H), lambda i: (0, 0)),
        ],
        out_specs=[
            pl.BlockSpec((bn, H), lambda i: (i, 0)),
            pl.BlockSpec((bn, H), lambda i: (i, 0)),
        ],
        out_shape=[
            jax.ShapeDtypeStruct((N, H), jnp.float32),
            jax.ShapeDtypeStruct((N, H), jnp.float32),
        ],
    )(sp, y, dinv, b, wa, wb, c0)


def _tc_edge_kernel(g_ref, ea_ref, sel_ref, we2_ref, c1_ref, wp1c_ref,
                    wp2_ref, bp2_ref, p_ref):
    er = jax.nn.relu(
        g_ref[...] + jnp.dot(ea_ref[...], we2_ref[...], preferred_element_type=jnp.float32))
    h = jax.nn.relu(
        jnp.dot(er, wp1c_ref[...], preferred_element_type=jnp.float32) + c1_ref[...])
    logit = jnp.dot(h, wp2_ref[...], preferred_element_type=jnp.float32) + bp2_ref[...]
    p = jax.nn.sigmoid(logit)
    p_ref[...] = jnp.where(sel_ref[...] != 0, p, -MNEG)


def _tc_edge(g, ea, sel, we2, c1, wp1c, wp2, bp2, h):
    be = 6400
    nb = E // (2 * be)
    off = h * nb
    return pl.pallas_call(
        _tc_edge_kernel,
        grid=(nb,),
        in_specs=[
            pl.BlockSpec((be, H), lambda i: (i, 0)),
            pl.BlockSpec((be, 16), lambda i: (i + off, 0)),
            pl.BlockSpec((be, 1), lambda i: (i + off, 0)),
            pl.BlockSpec((16, H), lambda i: (0, 0)),
            pl.BlockSpec((1, H), lambda i: (0, 0)),
            pl.BlockSpec((H, H), lambda i: (0, 0)),
            pl.BlockSpec((H, 1), lambda i: (0, 0)),
            pl.BlockSpec((1, 1), lambda i: (0, 0)),
        ],
        out_specs=pl.BlockSpec((be, 1), lambda i: (i, 0)),
        out_shape=jax.ShapeDtypeStruct((E // 2, 1), jnp.float32),
    )(g, ea, sel, we2, c1, wp1c, wp2, bp2)


def kernel(x, edge_attr, W0, b0, W1, b1, W2, b2, We, be, War, bar,
           Wp1, bp1, Wp2, bp2, Wg, edge_index, selection):
    src = edge_index[0].astype(jnp.int32)
    dst = edge_index[1].astype(jnp.int32)
    padn = EPAD - E
    padv = jnp.arange(padn, dtype=jnp.int32) % L
    src2d = jnp.concatenate([src, padv]).reshape(EPAD // L, L)
    dst2d = jnp.concatenate([dst, padv + N]).reshape(EPAD // L, L)

    zeros_n = jnp.zeros((NPAD,), jnp.float32)
    zeros_nh = jnp.zeros((NPAD, H), jnp.float32)
    ones_l = jnp.ones((L,), jnp.float32)

    we2, c0, c1 = _tc_prep(x, Wg, Wp1, War, We, be.reshape(1, H),
                           bar.reshape(1, H), bp1.reshape(1, H))

    _sc_degree, _sc_segsum, _sc_pair0, _sc_pair1 = _sc_kernels()

    degp = _sc_degree(dst2d, zeros_n, ones_l)
    degt = degp.reshape(NC, NPAD).T

    y0, dinv = _tc_first(x, W0, degt)
    sp0 = _sc_segsum(y0, src2d, dst2d, zeros_nh)
    y1 = _tc_layer(sp0, y0, dinv, b0.reshape(1, H), W1)
    sp1 = _sc_segsum(y1, src2d, dst2d, zeros_nh)
    y2 = _tc_layer(sp1, y1, dinv, b1.reshape(1, H), W2)
    sp2 = _sc_segsum(y2, src2d, dst2d, zeros_nh)

    wa = War[:H]
    wb = War[H:2 * H]
    a_tab, b_tab = _tc_last(sp2, y2, dinv, b2.reshape(1, H), wa, wb, c0)

    sel2d = jnp.concatenate(
        [selection.astype(jnp.int32), jnp.zeros((padn,), jnp.int32)]
    ).reshape(EPAD // L, L)
    trash2d = jnp.broadcast_to(
        jnp.arange(L, dtype=jnp.int32) + (EPAD // 2 - L), (HGPT, L))

    sel = selection.astype(jnp.int32).reshape(E, 1)
    g0 = _sc_pair0(a_tab, b_tab, src2d, dst2d, sel2d, trash2d)
    p0 = _tc_edge(g0, edge_attr, sel, we2, c1, Wp1[2 * H:], Wp2,
                  bp2.reshape(1, 1), 0)
    g1 = _sc_pair1(a_tab, b_tab, src2d, dst2d, sel2d, trash2d)
    p1 = _tc_edge(g1, edge_attr, sel, we2, c1, Wp1[2 * H:], Wp2,
                  bp2.reshape(1, 1), 1)
    return jnp.concatenate([p0, p1]).reshape(E)


# speedup vs baseline: 1.0635x; 1.0635x over previous
import functools

import jax
import jax.numpy as jnp
from jax import lax
from jax.experimental import pallas as pl
from jax.experimental.pallas import tpu as pltpu
from jax.experimental.pallas import tpu_sc as plsc

N = 10000
E = 320000
H = 128
MNEG = 1000.0

NC = 2
NS = 16
NW = NC * NS
L = 128
GPT = 80
EPAD = NW * GPT * L
NPAD = 10240
NSL = NPAD // NS
NBUF = 2
IGC = 40
PNBUF = 2


@functools.cache
def _sc_kernels():
    mesh = plsc.VectorSubcoreMesh(
        core_axis_name="c", subcore_axis_name="s", num_cores=NC, num_subcores=NS)
    deg = functools.partial(
        pl.kernel, mesh=mesh,
        out_type=jax.ShapeDtypeStruct((NC * NPAD,), jnp.float32),
        scratch_types=[
            pltpu.VMEM_SHARED((NPAD,), jnp.float32),
            pltpu.VMEM((GPT, L), jnp.int32),
            pltpu.VMEM((L,), jnp.float32),
            pltpu.VMEM((NSL,), jnp.float32),
        ],
    )(_sc_degree_body)
    seg = functools.partial(
        pl.kernel, mesh=mesh,
        out_type=jax.ShapeDtypeStruct((NC, NPAD, H), jnp.float32),
        scratch_types=[
            pltpu.VMEM_SHARED((NPAD, H), jnp.float32),
            pltpu.VMEM((IGC, L), jnp.int32),
            pltpu.VMEM((IGC, L), jnp.int32),
            [pltpu.VMEM((L, H), jnp.float32) for _ in range(NBUF)],
            [pltpu.SemaphoreType.DMA for _ in range(NBUF)],
        ],
    )(_sc_segsum_body)
    pair = functools.partial(
        pl.kernel, mesh=mesh,
        out_type=jax.ShapeDtypeStruct((EPAD, H), jnp.float32),
        scratch_types=[
            pltpu.VMEM((GPT, L), jnp.int32),
            pltpu.VMEM((GPT, L), jnp.int32),
            pltpu.VMEM((GPT, L), jnp.int32),
            pltpu.VMEM((GPT, L), jnp.int32),
            pltpu.VMEM((GPT, L), jnp.int32),
            pltpu.VMEM((GPT, L), jnp.int32),
            [pltpu.VMEM((L, H), jnp.float32) for _ in range(PNBUF)],
            [pltpu.VMEM((L, H), jnp.float32) for _ in range(PNBUF)],
            [pltpu.SemaphoreType.DMA for _ in range(PNBUF)],
            [pltpu.SemaphoreType.DMA for _ in range(PNBUF)],
        ],
        compiler_params=pltpu.CompilerParams(needs_layout_passes=False),
    )(_sc_pair_body)
    return deg, seg, pair


def _sc_degree_body(dst2d, zeros_n, ones_l, degp, dacc, didx, ones_v, vbuf):
    c = lax.axis_index("c")
    s = lax.axis_index("s")
    wid = c * NS + s
    pltpu.sync_copy(zeros_n.at[pl.ds(s * NSL, NSL)], vbuf)
    pltpu.sync_copy(vbuf, dacc.at[pl.ds(s * NSL, NSL)])
    pltpu.sync_copy(ones_l, ones_v)
    pltpu.sync_copy(dst2d.at[pl.ds(wid * GPT, GPT)], didx)
    plsc.subcore_barrier()

    @pl.loop(0, GPT)
    def _(j):
        pltpu.sync_copy(ones_v, dacc.at[didx.at[j]], add=True)

    plsc.subcore_barrier()
    pltpu.sync_copy(dacc.at[pl.ds(s * NSL, NSL)], vbuf)
    pltpu.sync_copy(vbuf, degp.at[pl.ds(c * NPAD + s * NSL, NSL)])


def _sc_segsum_body(y, src2d, dst2d, zeros_nh, acc_out, acc, sidx, didx, rows, sems):
    c = lax.axis_index("c")
    s = lax.axis_index("s")
    wid = c * NS + s
    pltpu.sync_copy(zeros_nh.at[pl.ds(s * NSL, NSL)], acc.at[pl.ds(s * NSL, NSL)])
    plsc.subcore_barrier()

    @pl.loop(0, GPT // IGC)
    def _(ic):
        base = wid * GPT + ic * IGC
        pltpu.sync_copy(src2d.at[pl.ds(base, IGC)], sidx)
        pltpu.sync_copy(dst2d.at[pl.ds(base, IGC)], didx)
        for b in range(NBUF):
            pltpu.async_copy(y.at[sidx.at[b]], rows[b], sems[b])

        @pl.loop(0, IGC, step=NBUF)
        def _(j):
            for b in range(NBUF):
                jj = j + b
                pltpu.make_async_copy(y.at[sidx.at[jj]], rows[b], sems[b]).wait()
                pltpu.sync_copy(rows[b], acc.at[didx.at[jj]], add=True)

                @pl.when(jj < IGC - NBUF)
                def _():
                    pltpu.async_copy(y.at[sidx.at[jj + NBUF]], rows[b], sems[b])

    plsc.subcore_barrier()
    pltpu.sync_copy(acc.at[pl.ds(s * NSL, NSL)], acc_out.at[c, pl.ds(s * NSL, NSL)])


def _sc_pair_body(a_tab, b_tab, src2d, dst2d, sel2d, trash2d, g_out,
                  sidx, didx, selv2, csrc, cdst, cpos,
                  bufa, bufb, sema, semb):
    c = lax.axis_index("c")
    s = lax.axis_index("s")
    wid = c * NS + s
    pltpu.sync_copy(src2d.at[pl.ds(wid * GPT, GPT)], sidx)
    pltpu.sync_copy(dst2d.at[pl.ds(wid * GPT, GPT)], didx)
    pltpu.sync_copy(sel2d.at[pl.ds(wid * GPT, GPT)], selv2)
    pltpu.sync_copy(src2d.at[pl.ds(wid * GPT, GPT)], csrc)
    pltpu.sync_copy(dst2d.at[pl.ds(wid * GPT, GPT)], cdst)
    pltpu.sync_copy(trash2d, cpos)

    lanes = lax.iota(jnp.int32, 16)

    @pl.loop(0, GPT, init_carry=jnp.zeros((16,), jnp.int32))
    def _compact(j, cnt):
        for k in range(L // 16):
            selv = selv2[j, pl.ds(k * 16, 16)]
            srcv = sidx[j, pl.ds(k * 16, 16)]
            dstv = didx[j, pl.ds(k * 16, 16)]
            posv = wid * (GPT * L) + j * L + k * 16 + lanes
            m = selv != 0
            pref = plsc.cumsum(selv)
            dest = cnt + pref - 1
            dr = lax.shift_right_logical(dest, 7)
            dl = lax.bitwise_and(dest, 127)
            plsc.store_scatter(csrc, [dr, dl], srcv, mask=m)
            plsc.store_scatter(cdst, [dr, dl], dstv, mask=m)
            plsc.store_scatter(cpos, [dr, dl], posv, mask=m)
            cnt = cnt + plsc.all_reduce_population_count(m)
        return cnt

    cnt = _compact[0]
    gc = lax.shift_right_logical(cnt + 127, 7)

    for b in range(PNBUF):
        @pl.when(b < gc)
        def _():
            pltpu.async_copy(a_tab.at[csrc.at[b]], bufa[b], sema[b])
            pltpu.async_copy(b_tab.at[cdst.at[b]], bufb[b], semb[b])

    gc_pad = lax.bitwise_and(gc + (PNBUF - 1), jnp.int32(-PNBUF))

    @pl.loop(0, gc_pad, step=PNBUF)
    def _(j):
        for b in range(PNBUF):
            jj = j + b

            @pl.when(jj < gc)
            def _():
                pltpu.make_async_copy(a_tab.at[csrc.at[jj]], bufa[b], sema[b]).wait()
                pltpu.make_async_copy(b_tab.at[cdst.at[jj]], bufb[b], semb[b]).wait()

                @pl.loop(0, L, unroll=4)
                def _(r):
                    for k in range(H // 16):
                        plsc.addupdate(bufa[b].at[r, pl.ds(k * 16, 16)],
                                       bufb[b][r, pl.ds(k * 16, 16)])

                pltpu.sync_copy(bufa[b], g_out.at[cpos.at[jj]])

                @pl.when(jj + PNBUF < gc)
                def _():
                    pltpu.async_copy(a_tab.at[csrc.at[jj + PNBUF]], bufa[b], sema[b])
                    pltpu.async_copy(b_tab.at[cdst.at[jj + PNBUF]], bufb[b], semb[b])


def _tc_prep_kernel(x_ref, wg_ref, wp1_ref, war_ref, we_ref, be_ref, bar_ref,
                    bp1_ref, we2_ref, c0_ref, c1_ref, acc):
    i = pl.program_id(0)

    @pl.when(i == 0)
    def _():
        acc[...] = jnp.zeros_like(acc)

    acc[...] += jnp.sum(x_ref[...], axis=0, keepdims=True)

    @pl.when(i == pl.num_programs(0) - 1)
    def _():
        wc = war_ref[2 * H:, :]
        we2_ref[...] = jnp.dot(we_ref[...], wc, preferred_element_type=jnp.float32)
        c0_ref[...] = jnp.dot(be_ref[...], wc, preferred_element_type=jnp.float32) + bar_ref[...]
        g = jnp.dot(acc[...] * (1.0 / N), wg_ref[...], preferred_element_type=jnp.float32)
        w1ab = wp1_ref[:H, :] + wp1_ref[H:2 * H, :]
        c1_ref[...] = jnp.dot(g, w1ab, preferred_element_type=jnp.float32) + bp1_ref[...]


def _tc_prep(x, wg, wp1, war, we, be, bar, bp1):
    nb = 10
    bn = N // nb
    return pl.pallas_call(
        _tc_prep_kernel,
        grid=(nb,),
        in_specs=[
            pl.BlockSpec((bn, H), lambda i: (i, 0)),
            pl.BlockSpec((H, H), lambda i: (0, 0)),
            pl.BlockSpec((3 * H, H), lambda i: (0, 0)),
            pl.BlockSpec((3 * H, H), lambda i: (0, 0)),
            pl.BlockSpec((16, H), lambda i: (0, 0)),
            pl.BlockSpec((1, H), lambda i: (0, 0)),
            pl.BlockSpec((1, H), lambda i: (0, 0)),
            pl.BlockSpec((1, H), lambda i: (0, 0)),
        ],
        out_specs=[
            pl.BlockSpec((16, H), lambda i: (0, 0)),
            pl.BlockSpec((1, H), lambda i: (0, 0)),
            pl.BlockSpec((1, H), lambda i: (0, 0)),
        ],
        out_shape=[
            jax.ShapeDtypeStruct((16, H), jnp.float32),
            jax.ShapeDtypeStruct((1, H), jnp.float32),
            jax.ShapeDtypeStruct((1, H), jnp.float32),
        ],
        scratch_shapes=[pltpu.VMEM((1, H), jnp.float32)],
    )(x, wg, wp1, war, we, be, bar, bp1)


def _tc_first_kernel(x_ref, w_ref, degt_ref, y_ref, dinv_ref):
    deg = 1.0 + degt_ref[:, 0:1] + degt_ref[:, 1:2]
    dinv = lax.rsqrt(deg)
    dinv_ref[...] = dinv
    y_ref[...] = jnp.dot(x_ref[...], w_ref[...], preferred_element_type=jnp.float32) * dinv


def _tc_first(x, w0, degt):
    nb = 10
    bn = N // nb
    return pl.pallas_call(
        _tc_first_kernel,
        grid=(nb,),
        in_specs=[
            pl.BlockSpec((bn, H), lambda i: (i, 0)),
            pl.BlockSpec((H, H), lambda i: (0, 0)),
            pl.BlockSpec((bn, 2), lambda i: (i, 0)),
        ],
        out_specs=[
            pl.BlockSpec((bn, H), lambda i: (i, 0)),
            pl.BlockSpec((bn, 1), lambda i: (i, 0)),
        ],
        out_shape=[
            jax.ShapeDtypeStruct((N, H), jnp.float32),
            jax.ShapeDtypeStruct((N, 1), jnp.float32),
        ],
    )(x, w0, degt)


def _tc_layer_kernel(sp_ref, y_ref, dinv_ref, b_ref, w_ref, ynext_ref):
    s = sp_ref[0] + sp_ref[1]
    dinv = dinv_ref[...]
    out = jax.nn.relu(dinv * (s + y_ref[...]) + b_ref[...])
    ynext_ref[...] = jnp.dot(out, w_ref[...], preferred_element_type=jnp.float32) * dinv


def _tc_layer(sp, y, dinv, b, wnext):
    nb = 10
    bn = N // nb
    return pl.pallas_call(
        _tc_layer_kernel,
        grid=(nb,),
        in_specs=[
            pl.BlockSpec((2, bn, H), lambda i: (0, i, 0)),
            pl.BlockSpec((bn, H), lambda i: (i, 0)),
            pl.BlockSpec((bn, 1), lambda i: (i, 0)),
            pl.BlockSpec((1, H), lambda i: (0, 0)),
            pl.BlockSpec((H, H), lambda i: (0, 0)),
        ],
        out_specs=pl.BlockSpec((bn, H), lambda i: (i, 0)),
        out_shape=jax.ShapeDtypeStruct((N, H), jnp.float32),
    )(sp, y, dinv, b, wnext)


def _tc_last_kernel(sp_ref, y_ref, dinv_ref, b_ref, wa_ref, wb_ref, c0_ref,
                    a_ref, bt_ref):
    s = sp_ref[0] + sp_ref[1]
    out = jax.nn.relu(dinv_ref[...] * (s + y_ref[...]) + b_ref[...])
    a_ref[...] = jnp.dot(out, wa_ref[...], preferred_element_type=jnp.float32) + c0_ref[...]
    bt_ref[...] = jnp.dot(out, wb_ref[...], preferred_element_type=jnp.float32)


def _tc_last(sp, y, dinv, b, wa, wb, c0):
    nb = 10
    bn = N // nb
    return pl.pallas_call(
        _tc_last_kernel,
        grid=(nb,),
        in_specs=[
            pl.BlockSpec((2, bn, H), lambda i: (0, i, 0)),
            pl.BlockSpec((bn, H), lambda i: (i, 0)),
            pl.BlockSpec((bn, 1), lambda i: (i, 0)),
            pl.BlockSpec((1, H), lambda i: (0, 0)),
            pl.BlockSpec((H, H), lambda i: (0, 0)),
            pl.BlockSpec((H, H), lambda i: (0, 0)),
            pl.BlockSpec((1, H), lambda i: (0, 0)),
        ],
        out_specs=[
            pl.BlockSpec((bn, H), lambda i: (i, 0)),
            pl.BlockSpec((bn, H), lambda i: (i, 0)),
        ],
        out_shape=[
            jax.ShapeDtypeStruct((N, H), jnp.float32),
            jax.ShapeDtypeStruct((N, H), jnp.float32),
        ],
    )(sp, y, dinv, b, wa, wb, c0)


def _tc_edge_kernel(g_ref, ea_ref, sel_ref, we2_ref, c1_ref, wp1c_ref,
                    wp2_ref, bp2_ref, p_ref):
    er = jax.nn.relu(
        g_ref[...] + jnp.dot(ea_ref[...], we2_ref[...], preferred_element_type=jnp.float32))
    h = jax.nn.relu(
        jnp.dot(er, wp1c_ref[...], preferred_element_type=jnp.float32) + c1_ref[...])
    logit = jnp.dot(h, wp2_ref[...], preferred_element_type=jnp.float32) + bp2_ref[...]
    p = jax.nn.sigmoid(logit)
    p_ref[...] = jnp.where(sel_ref[...] != 0, p, -MNEG)


def _tc_edge(g, ea, sel, we2, c1, wp1c, wp2, bp2):
    be = 6400
    nb = E // be
    return pl.pallas_call(
        _tc_edge_kernel,
        grid=(nb,),
        in_specs=[
            pl.BlockSpec((be, H), lambda i: (i, 0)),
            pl.BlockSpec((be, 16), lambda i: (i, 0)),
            pl.BlockSpec((be, 1), lambda i: (i, 0)),
            pl.BlockSpec((16, H), lambda i: (0, 0)),
            pl.BlockSpec((1, H), lambda i: (0, 0)),
            pl.BlockSpec((H, H), lambda i: (0, 0)),
            pl.BlockSpec((H, 1), lambda i: (0, 0)),
            pl.BlockSpec((1, 1), lambda i: (0, 0)),
        ],
        out_specs=pl.BlockSpec((be, 1), lambda i: (i, 0)),
        out_shape=jax.ShapeDtypeStruct((E, 1), jnp.float32),
    )(g, ea, sel, we2, c1, wp1c, wp2, bp2)


def kernel(x, edge_attr, W0, b0, W1, b1, W2, b2, We, be, War, bar,
           Wp1, bp1, Wp2, bp2, Wg, edge_index, selection):
    src = edge_index[0].astype(jnp.int32)
    dst = edge_index[1].astype(jnp.int32)
    padn = EPAD - E
    padv = jnp.arange(padn, dtype=jnp.int32) % L
    src2d = jnp.concatenate([src, padv]).reshape(EPAD // L, L)
    dst2d = jnp.concatenate([dst, padv + N]).reshape(EPAD // L, L)

    zeros_n = jnp.zeros((NPAD,), jnp.float32)
    zeros_nh = jnp.zeros((NPAD, H), jnp.float32)
    ones_l = jnp.ones((L,), jnp.float32)

    we2, c0, c1 = _tc_prep(x, Wg, Wp1, War, We, be.reshape(1, H),
                           bar.reshape(1, H), bp1.reshape(1, H))

    _sc_degree, _sc_segsum, _sc_pair = _sc_kernels()

    degp = _sc_degree(dst2d, zeros_n, ones_l)
    degt = degp.reshape(NC, NPAD).T

    y0, dinv = _tc_first(x, W0, degt)
    sp0 = _sc_segsum(y0, src2d, dst2d, zeros_nh)
    y1 = _tc_layer(sp0, y0, dinv, b0.reshape(1, H), W1)
    sp1 = _sc_segsum(y1, src2d, dst2d, zeros_nh)
    y2 = _tc_layer(sp1, y1, dinv, b1.reshape(1, H), W2)
    sp2 = _sc_segsum(y2, src2d, dst2d, zeros_nh)

    wa = War[:H]
    wb = War[H:2 * H]
    a_tab, b_tab = _tc_last(sp2, y2, dinv, b2.reshape(1, H), wa, wb, c0)

    sel2d = jnp.concatenate(
        [selection.astype(jnp.int32), jnp.zeros((padn,), jnp.int32)]
    ).reshape(EPAD // L, L)
    trash2d = jnp.broadcast_to(
        jnp.arange(L, dtype=jnp.int32) + (EPAD - L), (GPT, L))
    g_pairs = _sc_pair(a_tab, b_tab, src2d, dst2d, sel2d, trash2d)

    sel = selection.astype(jnp.int32).reshape(E, 1)
    p = _tc_edge(g_pairs, edge_attr, sel, we2, c1, Wp1[2 * H:], Wp2,
                 bp2.reshape(1, 1))
    return p.reshape(E)
